# baseline (speedup 1.0000x reference)
import jax
import jax.numpy as jnp
import numpy as np
from jax.experimental import pallas as pl

_BS = 1024
_NB = 30
_NP = 32
_K = 10
_HID = 128
_EMB = 64
_G = 32
_NEG = -1e30

_F = np.arange(_NP * _NP)
_SEL_I = (np.arange(_NP)[:, None] == (_F[None, :] // _NP)).astype(np.float32)
_SEL_J = (np.arange(_NP)[:, None] == (_F[None, :] % _NP)).astype(np.float32)
_DIAG = np.where((_F // _NP) == (_F % _NP), 1e10, 0.0).astype(np.float32)[None, :]
_C = np.arange(_NB * 128)
_PBIG = (np.arange(128)[:, None]
         == ((_C[None, :] % 128) // _NP) * _NP + _C[None, :] // 128
         ).astype(np.float32)


def _mask_kernel(xl0_ref, xl1_ref, seli_ref, selj_ref, diag_ref, pbig_ref,
                 biasp_ref):
    G, NP, NB, K = _G, _NP, _NB, _K
    def _exact_sel(v, sel):
        h = v.astype(jnp.bfloat16)
        r = v - h.astype(jnp.float32)
        m = r.astype(jnp.bfloat16)
        l = (r - m.astype(jnp.float32)).astype(jnp.bfloat16)
        return ((jnp.dot(h, sel, preferred_element_type=jnp.float32)
                 + jnp.dot(m, sel, preferred_element_type=jnp.float32))
                + jnp.dot(l, sel, preferred_element_type=jnp.float32))

    x0 = xl0_ref[:]
    x1 = xl1_ref[:]
    seli = seli_ref[:]
    selj = selj_ref[:]
    xi0 = _exact_sel(x0, seli)
    xj0 = _exact_sel(x0, selj)
    xi1 = _exact_sel(x1, seli)
    xj1 = _exact_sel(x1, selj)
    d2f = (xi0 - xj0) ** 2 + (xi1 - xj1) ** 2 + diag_ref[:]
    d2p = d2f.reshape(G * 8, 128)
    jp = jax.lax.broadcasted_iota(jnp.int32, (G * 8, 128), 1) % NP
    hi = d2p.astype(jnp.bfloat16)
    r1 = d2p - hi.astype(jnp.float32)
    mid = r1.astype(jnp.bfloat16)
    lo = (r1 - mid.astype(jnp.float32)).astype(jnp.bfloat16)
    Pb = pbig_ref[:]
    ckall = (jnp.dot(hi, Pb, preferred_element_type=jnp.float32)
             + jnp.dot(mid, Pb, preferred_element_type=jnp.float32)
             ) + jnp.dot(lo, Pb, preferred_element_type=jnp.float32)
    rankp = jnp.zeros((G * 8, 128), jnp.float32)
    for k in range(NB):
        ckb = ckall[:, k * 128:(k + 1) * 128]
        ltf = jnp.where(ckb < d2p, 1.0, 0.0)
        lef = jnp.where(ckb <= d2p, 1.0, 0.0)
        nself = jnp.where(jp == k, 0.0, 1.0)
        rankp = rankp + jnp.where(k < jp, lef, ltf) * nself
    biasp_ref[:] = jnp.where(rankp < K, 1.0, 0.0).astype(jnp.int8)


def _fused_kernel(inp4_ref, bias2_ref,
                  W1a_ref, b1a_ref, W1b_ref, b1b_ref,
                  emb_ref, Wc_ref, bc_ref,
                  Wm1_ref, bm1_ref, Wm2_ref, bm2_ref,
                  Ws1_ref, bs1_ref, Ws2_ref, bs2_ref,
                  out_ref):
    G, NP, NB, K = _G, _NP, _NB, _K
    R = G * NP

    inp4 = inp4_ref[:]
    lane4 = jax.lax.broadcasted_iota(jnp.int32, (R, 4), 1)
    t4 = jnp.where(lane4 < 2, inp4, jnp.tanh(inp4))
    h1 = jnp.tanh(jnp.dot(t4, W1a_ref[:], preferred_element_type=jnp.float32)
                  + b1a_ref[:])
    h = jnp.dot(h1, W1b_ref[:], preferred_element_type=jnp.float32) + b1b_ref[:]
    th = jnp.tanh(h)

    cf3 = jnp.dot(jnp.tanh(emb_ref[:]), Wc_ref[:],
                  preferred_element_type=jnp.float32)
    rid = jax.lax.broadcasted_iota(jnp.int32, (R, 3), 0)
    cid = jax.lax.broadcasted_iota(jnp.int32, (R, 3), 1)
    local = rid % NP
    onehot = jnp.where((local // 10 == cid) & (local < NB), 1.0, 0.0)
    cf = jnp.dot(onehot, cf3, preferred_element_type=jnp.float32) + bc_ref[:]
    tcf = jnp.tanh(cf)

    Wm1 = Wm1_ref[:]
    Wt_h = Wm1[0:128, :]
    Wt_c = Wm1[128:192, :]
    Wb_h = Wm1[192:320, :]
    Wb_c = Wm1[320:384, :]
    A = (jnp.dot(th, Wt_h - Wb_h, preferred_element_type=jnp.float32)
         + jnp.dot(tcf, Wt_c - Wb_c, preferred_element_type=jnp.float32)
         + bm1_ref[:])
    B = (jnp.dot(th, Wb_h, preferred_element_type=jnp.float32)
         + jnp.dot(tcf, Wb_c, preferred_element_type=jnp.float32))
    A3 = A.reshape(G, NP, _HID)
    B3 = B.reshape(G, NP, _HID)

    bias2 = bias2_ref[:]
    Wm2 = Wm2_ref[:]
    acc = jnp.full((G, NP, _HID), _NEG, jnp.float32)
    for j in range(NB):
        Bj = B3[:, j:j + 1, :]
        tj = jnp.tanh(A3 + Bj).reshape(R, _HID)
        ej = jnp.dot(tj, Wm2, preferred_element_type=jnp.float32).reshape(G, NP, _HID)
        mj = bias2[:, j:j + 1].astype(jnp.float32) * 1e30 - 1e30
        bj = jnp.broadcast_to(mj, (R, _HID)).reshape(G, NP, _HID)
        acc = jnp.maximum(acc, ej + bj)
    xh = jnp.tanh(acc.reshape(R, _HID) + bm2_ref[:])

    o1 = jnp.tanh(jnp.dot(xh, Ws1_ref[:], preferred_element_type=jnp.float32)
                  + bs1_ref[:])
    o = jnp.dot(o1, Ws2_ref[:], preferred_element_type=jnp.float32) + bs2_ref[:]
    mu = jnp.tanh(o[:, 0:2]) + t4[:, 2:4]
    ls = jnp.tanh(o[:, 2:4])
    std = jnp.exp(-5.0 + 3.5 * (ls + 1.0))
    out_ref[:, 0:2] = mu
    out_ref[:, 2:4] = std


def kernel(state_inp, tar_scores, W1a, b1a, W1b, b1b, emb, Wc, bc,
           Wm1, bm1, Wm2, bm2, Ws1, bs1, Ws2, bs2):
    BS, NB, NP, G = _BS, _NB, _NP, _G
    x = state_inp.reshape(BS, NB, 2)
    ts = tar_scores.reshape(BS, NB, 2)
    inp4 = jnp.zeros((BS, NP, 4), jnp.float32)
    inp4 = inp4.at[:, :NB, 0:2].set(x).at[:, :NB, 2:4].set(ts)
    inp4 = inp4.reshape(BS * NP, 4)
    xl0 = jnp.full((BS, NP), 1e15, jnp.float32).at[:, :NB].set(x[:, :, 0])
    xl1 = jnp.full((BS, NP), 1e15, jnp.float32).at[:, :NB].set(x[:, :, 1])

    seli = jnp.asarray(_SEL_I, dtype=jnp.bfloat16)
    selj = jnp.asarray(_SEL_J, dtype=jnp.bfloat16)
    diag = jnp.asarray(_DIAG)
    pbig = jnp.asarray(_PBIG, dtype=jnp.bfloat16)

    row = lambda v: v.reshape(1, -1)
    grid = (BS // G,)

    def full(a):
        nd = a.ndim
        return pl.BlockSpec(a.shape, lambda i, _nd=nd: (0,) * _nd)

    biasp = pl.pallas_call(
        _mask_kernel,
        grid=grid,
        in_specs=[
            pl.BlockSpec((G, NP), lambda i: (i, 0)),
            pl.BlockSpec((G, NP), lambda i: (i, 0)),
            full(seli), full(selj), full(diag), full(pbig),
        ],
        out_specs=pl.BlockSpec((G * 8, 128), lambda i: (i, 0)),
        out_shape=jax.ShapeDtypeStruct((BS * 8, 128), jnp.int8),
    )(xl0, xl1, seli, selj, diag, pbig)
    bias2 = biasp.reshape(BS * NP, NP)

    res = pl.pallas_call(
        _fused_kernel,
        grid=grid,
        in_specs=[
            pl.BlockSpec((G * NP, 4), lambda i: (i, 0)),
            pl.BlockSpec((G * NP, NP), lambda i: (i, 0)),
            full(W1a), full(row(b1a)), full(W1b), full(row(b1b)),
            full(emb), full(Wc), full(row(bc)),
            full(Wm1), full(row(bm1)), full(Wm2), full(row(bm2)),
            full(Ws1), full(row(bs1)), full(Ws2), full(row(bs2)),
        ],
        out_specs=pl.BlockSpec((G * NP, 4), lambda i: (i, 0)),
        out_shape=jax.ShapeDtypeStruct((BS * NP, 4), jnp.float32),
    )(inp4, bias2,
      W1a, row(b1a), W1b, row(b1b), emb, Wc, row(bc),
      Wm1, row(bm1), Wm2, row(bm2), Ws1, row(bs1), Ws2, row(bs2))

    res = res.reshape(BS, NP, 4)[:, :NB, :]
    mu = res[:, :, 0:2].reshape(BS, 2 * NB)
    std = res[:, :, 2:4].reshape(BS, 2 * NB)
    return jnp.concatenate([mu, std], axis=-1)


# speedup vs baseline: 1.1243x; 1.1243x over previous
import jax
import jax.numpy as jnp
import numpy as np
from jax.experimental import pallas as pl

_BS = 1024
_NB = 30
_NP = 32
_K = 10
_HID = 128
_EMB = 64
_G = 32
_NEG = -1e30

_F = np.arange(_NP * _NP)
_SEL_I = (np.arange(_NP)[:, None] == (_F[None, :] // _NP)).astype(np.float32)
_SEL_J = (np.arange(_NP)[:, None] == (_F[None, :] % _NP)).astype(np.float32)
_DIAG = np.where((_F // _NP) == (_F % _NP), 1e10, 0.0).astype(np.float32)[None, :]
_C = np.arange(_NB * 128)
_PBIG = (np.arange(128)[:, None]
         == ((_C[None, :] % 128) // _NP) * _NP + _C[None, :] // 128
         ).astype(np.float32)


def _mask_kernel(xl0_ref, xl1_ref, seli_ref, selj_ref, diag_ref, pbig_ref,
                 biasp_ref):
    G, NP, NB, K = _G, _NP, _NB, _K
    def _exact_sel(v, sel):
        h = v.astype(jnp.bfloat16)
        r = v - h.astype(jnp.float32)
        m = r.astype(jnp.bfloat16)
        l = (r - m.astype(jnp.float32)).astype(jnp.bfloat16)
        return ((jnp.dot(h, sel, preferred_element_type=jnp.float32)
                 + jnp.dot(m, sel, preferred_element_type=jnp.float32))
                + jnp.dot(l, sel, preferred_element_type=jnp.float32))

    x0 = xl0_ref[:]
    x1 = xl1_ref[:]
    seli = seli_ref[:]
    selj = selj_ref[:]
    xi0 = _exact_sel(x0, seli)
    xj0 = _exact_sel(x0, selj)
    xi1 = _exact_sel(x1, seli)
    xj1 = _exact_sel(x1, selj)
    d2f = (xi0 - xj0) ** 2 + (xi1 - xj1) ** 2 + diag_ref[:]
    d2p = d2f.reshape(G * 8, 128)
    jp = jax.lax.broadcasted_iota(jnp.int32, (G * 8, 128), 1) % NP
    hi = d2p.astype(jnp.bfloat16)
    r1 = d2p - hi.astype(jnp.float32)
    mid = r1.astype(jnp.bfloat16)
    lo = (r1 - mid.astype(jnp.float32)).astype(jnp.bfloat16)
    Pb = pbig_ref[:]
    ckall = (jnp.dot(hi, Pb, preferred_element_type=jnp.float32)
             + jnp.dot(mid, Pb, preferred_element_type=jnp.float32)
             ) + jnp.dot(lo, Pb, preferred_element_type=jnp.float32)
    rankp = jnp.zeros((G * 8, 128), jnp.float32)
    for k in range(NB):
        ckb = ckall[:, k * 128:(k + 1) * 128]
        ltf = jnp.where(ckb < d2p, 1.0, 0.0)
        lef = jnp.where(ckb <= d2p, 1.0, 0.0)
        nself = jnp.where(jp == k, 0.0, 1.0)
        rankp = rankp + jnp.where(k < jp, lef, ltf) * nself
    biasp_ref[:] = jnp.where(rankp < K, 0.0, _NEG).astype(jnp.bfloat16)


def _fused_kernel(inp4_ref, bias2_ref,
                  W1a_ref, b1a_ref, W1b_ref, b1b_ref,
                  emb_ref, Wc_ref, bc_ref,
                  Wm1_ref, bm1_ref, Wm2_ref, bm2_ref,
                  Ws1_ref, bs1_ref, Ws2_ref, bs2_ref,
                  out_ref):
    G, NP, NB, K = _G, _NP, _NB, _K
    R = G * NP

    inp4 = inp4_ref[:]
    lane4 = jax.lax.broadcasted_iota(jnp.int32, (R, 4), 1)
    t4 = jnp.where(lane4 < 2, inp4, jnp.tanh(inp4))
    h1 = jnp.tanh(jnp.dot(t4, W1a_ref[:], preferred_element_type=jnp.float32)
                  + b1a_ref[:])
    h = jnp.dot(h1, W1b_ref[:], preferred_element_type=jnp.float32) + b1b_ref[:]
    th = jnp.tanh(h)

    cf3 = jnp.dot(jnp.tanh(emb_ref[:]), Wc_ref[:],
                  preferred_element_type=jnp.float32)
    rid = jax.lax.broadcasted_iota(jnp.int32, (R, 3), 0)
    cid = jax.lax.broadcasted_iota(jnp.int32, (R, 3), 1)
    local = rid % NP
    onehot = jnp.where((local // 10 == cid) & (local < NB), 1.0, 0.0)
    cf = jnp.dot(onehot, cf3, preferred_element_type=jnp.float32) + bc_ref[:]
    tcf = jnp.tanh(cf)

    Wm1 = Wm1_ref[:]
    Wt_h = Wm1[0:128, :]
    Wt_c = Wm1[128:192, :]
    Wb_h = Wm1[192:320, :]
    Wb_c = Wm1[320:384, :]
    A = (jnp.dot(th, Wt_h - Wb_h, preferred_element_type=jnp.float32)
         + jnp.dot(tcf, Wt_c - Wb_c, preferred_element_type=jnp.float32)
         + bm1_ref[:])
    B = (jnp.dot(th, Wb_h, preferred_element_type=jnp.float32)
         + jnp.dot(tcf, Wb_c, preferred_element_type=jnp.float32))
    A3 = A.reshape(G, NP, _HID)
    B3 = B.reshape(G, NP, _HID)

    bias2 = bias2_ref[:]
    Wm2 = Wm2_ref[:].astype(jnp.bfloat16)
    acc = jnp.full((G, NP, _HID), _NEG, jnp.float32)
    for j in range(NB):
        Bj = B3[:, j:j + 1, :]
        tj = jnp.tanh(A3 + Bj).reshape(R, _HID).astype(jnp.bfloat16)
        ej = jnp.dot(tj, Wm2, preferred_element_type=jnp.float32).reshape(G, NP, _HID)
        bj = jnp.broadcast_to(bias2[:, j:j + 1].astype(jnp.float32), (R, _HID)).reshape(G, NP, _HID)
        acc = jnp.maximum(acc, ej + bj)
    xh = jnp.tanh(acc.reshape(R, _HID) + bm2_ref[:])

    o1 = jnp.tanh(jnp.dot(xh, Ws1_ref[:], preferred_element_type=jnp.float32)
                  + bs1_ref[:])
    o = jnp.dot(o1, Ws2_ref[:], preferred_element_type=jnp.float32) + bs2_ref[:]
    mu = jnp.tanh(o[:, 0:2]) + t4[:, 2:4]
    ls = jnp.tanh(o[:, 2:4])
    std = jnp.exp(-5.0 + 3.5 * (ls + 1.0))
    out_ref[:, 0:2] = mu
    out_ref[:, 2:4] = std


def kernel(state_inp, tar_scores, W1a, b1a, W1b, b1b, emb, Wc, bc,
           Wm1, bm1, Wm2, bm2, Ws1, bs1, Ws2, bs2):
    BS, NB, NP, G = _BS, _NB, _NP, _G
    x = state_inp.reshape(BS, NB, 2)
    ts = tar_scores.reshape(BS, NB, 2)
    inp4 = jnp.zeros((BS, NP, 4), jnp.float32)
    inp4 = inp4.at[:, :NB, 0:2].set(x).at[:, :NB, 2:4].set(ts)
    inp4 = inp4.reshape(BS * NP, 4)
    xl0 = jnp.full((BS, NP), 1e15, jnp.float32).at[:, :NB].set(x[:, :, 0])
    xl1 = jnp.full((BS, NP), 1e15, jnp.float32).at[:, :NB].set(x[:, :, 1])

    seli = jnp.asarray(_SEL_I, dtype=jnp.bfloat16)
    selj = jnp.asarray(_SEL_J, dtype=jnp.bfloat16)
    diag = jnp.asarray(_DIAG)
    pbig = jnp.asarray(_PBIG, dtype=jnp.bfloat16)

    row = lambda v: v.reshape(1, -1)
    grid = (BS // G,)

    def full(a):
        nd = a.ndim
        return pl.BlockSpec(a.shape, lambda i, _nd=nd: (0,) * _nd)

    biasp = pl.pallas_call(
        _mask_kernel,
        grid=grid,
        in_specs=[
            pl.BlockSpec((G, NP), lambda i: (i, 0)),
            pl.BlockSpec((G, NP), lambda i: (i, 0)),
            full(seli), full(selj), full(diag), full(pbig),
        ],
        out_specs=pl.BlockSpec((G * 8, 128), lambda i: (i, 0)),
        out_shape=jax.ShapeDtypeStruct((BS * 8, 128), jnp.bfloat16),
    )(xl0, xl1, seli, selj, diag, pbig)
    bias2 = biasp.reshape(BS * NP, NP)

    res = pl.pallas_call(
        _fused_kernel,
        grid=grid,
        in_specs=[
            pl.BlockSpec((G * NP, 4), lambda i: (i, 0)),
            pl.BlockSpec((G * NP, NP), lambda i: (i, 0)),
            full(W1a), full(row(b1a)), full(W1b), full(row(b1b)),
            full(emb), full(Wc), full(row(bc)),
            full(Wm1), full(row(bm1)), full(Wm2), full(row(bm2)),
            full(Ws1), full(row(bs1)), full(Ws2), full(row(bs2)),
        ],
        out_specs=pl.BlockSpec((G * NP, 4), lambda i: (i, 0)),
        out_shape=jax.ShapeDtypeStruct((BS * NP, 4), jnp.float32),
    )(inp4, bias2,
      W1a, row(b1a), W1b, row(b1b), emb, Wc, row(bc),
      Wm1, row(bm1), Wm2, row(bm2), Ws1, row(bs1), Ws2, row(bs2))

    res = res.reshape(BS, NP, 4)[:, :NB, :]
    mu = res[:, :, 0:2].reshape(BS, 2 * NB)
    std = res[:, :, 2:4].reshape(BS, 2 * NB)
    return jnp.concatenate([mu, std], axis=-1)


# speedup vs baseline: 1.2253x; 1.0898x over previous
import jax
import jax.numpy as jnp
import numpy as np
from jax.experimental import pallas as pl

_BS = 1024
_NB = 30
_NP = 32
_K = 10
_HID = 128
_EMB = 64
_G = 32
_NEG = -1e30

_F = np.arange(_NP * _NP)
_SEL_I = (np.arange(_NP)[:, None] == (_F[None, :] // _NP)).astype(np.float32)
_SEL_J = (np.arange(_NP)[:, None] == (_F[None, :] % _NP)).astype(np.float32)
_DIAG = np.where((_F // _NP) == (_F % _NP), 1e10, 0.0).astype(np.float32)[None, :]
_C = np.arange(_NB * 128)
_PBIG = (np.arange(128)[:, None]
         == ((_C[None, :] % 128) // _NP) * _NP + _C[None, :] // 128
         ).astype(np.float32)


def _mask_kernel(xl0_ref, xl1_ref, seli_ref, selj_ref, diag_ref, pbig_ref,
                 biasp_ref):
    G, NP, NB, K = _G, _NP, _NB, _K
    def _exact_sel(v, sel):
        h = v.astype(jnp.bfloat16)
        r = v - h.astype(jnp.float32)
        m = r.astype(jnp.bfloat16)
        l = (r - m.astype(jnp.float32)).astype(jnp.bfloat16)
        return ((jnp.dot(h, sel, preferred_element_type=jnp.float32)
                 + jnp.dot(m, sel, preferred_element_type=jnp.float32))
                + jnp.dot(l, sel, preferred_element_type=jnp.float32))

    x0 = xl0_ref[:]
    x1 = xl1_ref[:]
    seli = seli_ref[:]
    selj = selj_ref[:]
    xi0 = _exact_sel(x0, seli)
    xj0 = _exact_sel(x0, selj)
    xi1 = _exact_sel(x1, seli)
    xj1 = _exact_sel(x1, selj)
    d2f = (xi0 - xj0) ** 2 + (xi1 - xj1) ** 2 + diag_ref[:]
    d2p = d2f.reshape(G * 8, 128)
    jp = jax.lax.broadcasted_iota(jnp.int32, (G * 8, 128), 1) % NP
    hi = d2p.astype(jnp.bfloat16)
    r1 = d2p - hi.astype(jnp.float32)
    mid = r1.astype(jnp.bfloat16)
    lo = (r1 - mid.astype(jnp.float32)).astype(jnp.bfloat16)
    Pb = pbig_ref[:]
    ckall = (jnp.dot(hi, Pb, preferred_element_type=jnp.float32)
             + jnp.dot(mid, Pb, preferred_element_type=jnp.float32)
             ) + jnp.dot(lo, Pb, preferred_element_type=jnp.float32)
    rankp = jnp.zeros((G * 8, 128), jnp.float32)
    for k in range(NB):
        ckb = ckall[:, k * 128:(k + 1) * 128]
        ltf = jnp.where(ckb < d2p, 1.0, 0.0)
        lef = jnp.where(ckb <= d2p, 1.0, 0.0)
        nself = jnp.where(jp == k, 0.0, 1.0)
        rankp = rankp + jnp.where(k < jp, lef, ltf) * nself
    biasp_ref[:] = jnp.where(rankp < K, 0.0, _NEG).astype(jnp.bfloat16)


def _fused_kernel(inp4_ref, bias2_ref,
                  W1a_ref, b1a_ref, W1b_ref, b1b_ref,
                  emb_ref, Wc_ref, bc_ref,
                  Wm1_ref, bm1_ref, Wm2_ref, bm2_ref,
                  Ws1_ref, bs1_ref, Ws2_ref, bs2_ref,
                  out_ref):
    G, NP, NB, K = _G, _NP, _NB, _K
    R = G * NP

    inp4 = inp4_ref[:]
    lane4 = jax.lax.broadcasted_iota(jnp.int32, (R, 4), 1)
    t4 = jnp.where(lane4 < 2, inp4, jnp.tanh(inp4))
    h1 = jnp.tanh(jnp.dot(t4, W1a_ref[:], preferred_element_type=jnp.float32)
                  + b1a_ref[:])
    h = jnp.dot(h1, W1b_ref[:], preferred_element_type=jnp.float32) + b1b_ref[:]
    th = jnp.tanh(h)

    cf3 = jnp.dot(jnp.tanh(emb_ref[:]), Wc_ref[:],
                  preferred_element_type=jnp.float32)
    rid = jax.lax.broadcasted_iota(jnp.int32, (R, 3), 0)
    cid = jax.lax.broadcasted_iota(jnp.int32, (R, 3), 1)
    local = rid % NP
    onehot = jnp.where((local // 10 == cid) & (local < NB), 1.0, 0.0)
    cf = jnp.dot(onehot, cf3, preferred_element_type=jnp.float32) + bc_ref[:]
    tcf = jnp.tanh(cf)

    Wm1 = Wm1_ref[:]
    Wt_h = Wm1[0:128, :]
    Wt_c = Wm1[128:192, :]
    Wb_h = Wm1[192:320, :]
    Wb_c = Wm1[320:384, :]
    A = (jnp.dot(th, Wt_h - Wb_h, preferred_element_type=jnp.float32)
         + jnp.dot(tcf, Wt_c - Wb_c, preferred_element_type=jnp.float32)
         + bm1_ref[:])
    B = (jnp.dot(th, Wb_h, preferred_element_type=jnp.float32)
         + jnp.dot(tcf, Wb_c, preferred_element_type=jnp.float32))
    A3 = A.reshape(G, NP, _HID).astype(jnp.bfloat16)
    B3 = B.reshape(G, NP, _HID).astype(jnp.bfloat16)

    bias2 = bias2_ref[:]
    Wm2 = Wm2_ref[:].astype(jnp.bfloat16)
    acc = jnp.full((G, NP, _HID), _NEG, jnp.float32)
    for j in range(NB):
        Bj = B3[:, j:j + 1, :]
        tj = jnp.tanh(A3 + Bj).reshape(R, _HID)
        ej = jnp.dot(tj, Wm2, preferred_element_type=jnp.float32).reshape(G, NP, _HID)
        bj = jnp.broadcast_to(bias2[:, j:j + 1].astype(jnp.float32), (R, _HID)).reshape(G, NP, _HID)
        acc = jnp.maximum(acc, ej + bj)
    xh = jnp.tanh(acc.reshape(R, _HID) + bm2_ref[:])

    o1 = jnp.tanh(jnp.dot(xh, Ws1_ref[:], preferred_element_type=jnp.float32)
                  + bs1_ref[:])
    o = jnp.dot(o1, Ws2_ref[:], preferred_element_type=jnp.float32) + bs2_ref[:]
    mu = jnp.tanh(o[:, 0:2]) + t4[:, 2:4]
    ls = jnp.tanh(o[:, 2:4])
    std = jnp.exp(-5.0 + 3.5 * (ls + 1.0))
    out_ref[:, 0:2] = mu
    out_ref[:, 2:4] = std


def kernel(state_inp, tar_scores, W1a, b1a, W1b, b1b, emb, Wc, bc,
           Wm1, bm1, Wm2, bm2, Ws1, bs1, Ws2, bs2):
    BS, NB, NP, G = _BS, _NB, _NP, _G
    x = state_inp.reshape(BS, NB, 2)
    ts = tar_scores.reshape(BS, NB, 2)
    inp4 = jnp.zeros((BS, NP, 4), jnp.float32)
    inp4 = inp4.at[:, :NB, 0:2].set(x).at[:, :NB, 2:4].set(ts)
    inp4 = inp4.reshape(BS * NP, 4)
    xl0 = jnp.full((BS, NP), 1e15, jnp.float32).at[:, :NB].set(x[:, :, 0])
    xl1 = jnp.full((BS, NP), 1e15, jnp.float32).at[:, :NB].set(x[:, :, 1])

    seli = jnp.asarray(_SEL_I, dtype=jnp.bfloat16)
    selj = jnp.asarray(_SEL_J, dtype=jnp.bfloat16)
    diag = jnp.asarray(_DIAG)
    pbig = jnp.asarray(_PBIG, dtype=jnp.bfloat16)

    row = lambda v: v.reshape(1, -1)
    grid = (BS // G,)

    def full(a):
        nd = a.ndim
        return pl.BlockSpec(a.shape, lambda i, _nd=nd: (0,) * _nd)

    biasp = pl.pallas_call(
        _mask_kernel,
        grid=grid,
        in_specs=[
            pl.BlockSpec((G, NP), lambda i: (i, 0)),
            pl.BlockSpec((G, NP), lambda i: (i, 0)),
            full(seli), full(selj), full(diag), full(pbig),
        ],
        out_specs=pl.BlockSpec((G * 8, 128), lambda i: (i, 0)),
        out_shape=jax.ShapeDtypeStruct((BS * 8, 128), jnp.bfloat16),
    )(xl0, xl1, seli, selj, diag, pbig)
    bias2 = biasp.reshape(BS * NP, NP)

    res = pl.pallas_call(
        _fused_kernel,
        grid=grid,
        in_specs=[
            pl.BlockSpec((G * NP, 4), lambda i: (i, 0)),
            pl.BlockSpec((G * NP, NP), lambda i: (i, 0)),
            full(W1a), full(row(b1a)), full(W1b), full(row(b1b)),
            full(emb), full(Wc), full(row(bc)),
            full(Wm1), full(row(bm1)), full(Wm2), full(row(bm2)),
            full(Ws1), full(row(bs1)), full(Ws2), full(row(bs2)),
        ],
        out_specs=pl.BlockSpec((G * NP, 4), lambda i: (i, 0)),
        out_shape=jax.ShapeDtypeStruct((BS * NP, 4), jnp.float32),
    )(inp4, bias2,
      W1a, row(b1a), W1b, row(b1b), emb, Wc, row(bc),
      Wm1, row(bm1), Wm2, row(bm2), Ws1, row(bs1), Ws2, row(bs2))

    res = res.reshape(BS, NP, 4)[:, :NB, :]
    mu = res[:, :, 0:2].reshape(BS, 2 * NB)
    std = res[:, :, 2:4].reshape(BS, 2 * NB)
    return jnp.concatenate([mu, std], axis=-1)
